# Initial kernel scaffold; baseline (speedup 1.0000x reference)
import jax
import jax.numpy as jnp
from jax.experimental import pallas as pl


def kernel(state, option, linear1, linear2, mean_linear, log_std_linear, mean_bias, log_std_bias):
    raise NotImplementedError("write your pallas kernel here")


# speedup vs baseline: 6.2765x; 6.2765x over previous
import jax
import jax.numpy as jnp
from jax.experimental import pallas as pl
from jax.experimental.pallas import tpu as pltpu


def _moe_kern(opt_ref, state_ref, l1_ref, l2_ref, ml_ref, lsl_ref, mb_ref,
              lsb_ref, mean_out, ls_out):
    o = pl.program_id(0)
    x = state_ref[...]
    h1 = jnp.maximum(
        jnp.dot(x, l1_ref[0], preferred_element_type=jnp.float32), 0.0)
    h2 = jnp.maximum(
        jnp.dot(h1, l2_ref[0], preferred_element_type=jnp.float32), 0.0)
    mean_o = jnp.dot(h2, ml_ref[0], preferred_element_type=jnp.float32)
    mean_o = mean_o + mb_ref[0]
    ls_o = jnp.dot(h2, lsl_ref[0], preferred_element_type=jnp.float32)
    ls_o = ls_o + lsb_ref[0]
    ls_o = jnp.clip(ls_o, -20.0, 2.0)
    mask = opt_ref[...] == o

    @pl.when(o == 0)
    def _():
        mean_out[...] = jnp.zeros_like(mean_out)
        ls_out[...] = jnp.zeros_like(ls_out)

    mean_out[...] = jnp.where(mask, mean_o, mean_out[...])
    ls_out[...] = jnp.where(mask, ls_o, ls_out[...])


def kernel(state, option, linear1, linear2, mean_linear, log_std_linear,
           mean_bias, log_std_bias):
    B, I = state.shape
    O, _, H = linear1.shape
    A = mean_bias.shape[1]
    Hc = linear2.shape[2]
    opt = jnp.broadcast_to(option.astype(jnp.int32).reshape(B, 1), (B, A))
    mb3 = mean_bias.reshape(O, 1, A)
    lsb3 = log_std_bias.reshape(O, 1, A)

    out_shape = (jax.ShapeDtypeStruct((B, A), jnp.float32),
                 jax.ShapeDtypeStruct((B, A), jnp.float32))
    const2 = lambda o: (0, 0)
    mean, log_std = pl.pallas_call(
        _moe_kern,
        grid=(O,),
        in_specs=[
            pl.BlockSpec((B, A), const2),
            pl.BlockSpec((B, I), const2),
            pl.BlockSpec((1, I, H), lambda o: (o, 0, 0)),
            pl.BlockSpec((1, H, Hc), lambda o: (o, 0, 0)),
            pl.BlockSpec((1, Hc, A), lambda o: (o, 0, 0)),
            pl.BlockSpec((1, Hc, A), lambda o: (o, 0, 0)),
            pl.BlockSpec((1, 1, A), lambda o: (o, 0, 0)),
            pl.BlockSpec((1, 1, A), lambda o: (o, 0, 0)),
        ],
        out_specs=(pl.BlockSpec((B, A), const2),
                   pl.BlockSpec((B, A), const2)),
        out_shape=out_shape,
        compiler_params=pltpu.CompilerParams(
            dimension_semantics=("arbitrary",)),
    )(opt, state, linear1, linear2, mean_linear, log_std_linear,
      mb3, lsb3)
    return (mean, log_std)
